# baseline (speedup 1.0000x reference)
import functools

import jax
import jax.numpy as jnp
from jax import lax
from jax.experimental import pallas as pl
from jax.experimental.pallas import tpu as pltpu
from jax.experimental.pallas import tpu_sc as plsc

D = 128
NW = 32
CHUNK = 128
NBUF = 5
LAG = 2


def _make_gather(nch):
    rpw = nch * CHUNK
    mesh = plsc.VectorSubcoreMesh(core_axis_name="c", subcore_axis_name="s")

    @functools.partial(
        pl.kernel,
        out_type=jax.ShapeDtypeStruct((NW * rpw, D), jnp.float32),
        mesh=mesh,
        scratch_types=[
            pltpu.VMEM((nch, CHUNK), jnp.int32),
            pltpu.VMEM((NBUF, CHUNK, D), jnp.float32),
        ] + [pltpu.SemaphoreType.DMA] * (2 * NBUF),
    )
    def gather(table_hbm, idx_hbm, out_hbm, idx_v, bufs, *sems):
        gsem = sems[:NBUF]
        wsem = sems[NBUF:]
        wid = lax.axis_index("s") * 2 + lax.axis_index("c")
        base = wid * rpw
        pltpu.sync_copy(idx_hbm.at[wid], idx_v)

        for b in range(LAG):
            pltpu.async_copy(table_hbm.at[idx_v.at[b]], bufs.at[b], gsem[b])

        @pl.loop(0, nch, step=NBUF)
        def _grp(g):
            for b in range(NBUF):
                j = g + b
                jg = j + LAG
                bg = (b + LAG) % NBUF

                @pl.when(jg < nch)
                def _issue_gather():
                    @pl.when(jg >= NBUF)
                    def _wait_write():
                        pltpu.make_async_copy(
                            bufs.at[bg],
                            out_hbm.at[pl.ds(base + (jg - NBUF) * CHUNK, CHUNK)],
                            wsem[bg],
                        ).wait()

                    pltpu.async_copy(
                        table_hbm.at[idx_v.at[jg]], bufs.at[bg], gsem[bg]
                    )

                pltpu.make_async_copy(
                    table_hbm.at[idx_v.at[j]], bufs.at[b], gsem[b]
                ).wait()
                pltpu.async_copy(
                    bufs.at[b], out_hbm.at[pl.ds(base + j * CHUNK, CHUNK)],
                    wsem[b],
                )

        for b in range(NBUF):
            j = nch - NBUF + b
            pltpu.make_async_copy(
                bufs.at[b], out_hbm.at[pl.ds(base + j * CHUNK, CHUNK)],
                wsem[b],
            ).wait()

    return gather


def kernel(x, word_embed):
    bat, hist = x.shape
    bh = bat * hist
    assert bh % (NW * CHUNK) == 0
    nch = bh // (NW * CHUNK)
    assert nch % NBUF == 0 and LAG < NBUF
    idx3 = x.reshape(NW, nch, CHUNK).astype(jnp.int32)
    out = _make_gather(nch)(word_embed, idx3)
    return out.reshape(bat, hist, D)


# speedup vs baseline: 1.7827x; 1.7827x over previous
import functools

import jax
import jax.numpy as jnp
from jax import lax
from jax.experimental import pallas as pl
from jax.experimental.pallas import tpu as pltpu
from jax.experimental.pallas import tpu_sc as plsc

D = 128
NW = 32
BPC = 2
NBUF = 4
LAG = 2


def _make_gather(bat, hist):
    rows = BPC * hist
    bpw = bat // NW
    nch = bpw // BPC
    mesh = plsc.VectorSubcoreMesh(core_axis_name="c", subcore_axis_name="s")

    @functools.partial(
        pl.kernel,
        out_type=jax.ShapeDtypeStruct((bat, hist, D), jnp.float32),
        mesh=mesh,
        compiler_params=pltpu.CompilerParams(use_tc_tiling_on_sc=True),
        scratch_types=[
            pltpu.VMEM((nch, rows), jnp.int32),
            pltpu.VMEM((NBUF, rows, D), jnp.float32),
        ] + [pltpu.SemaphoreType.DMA] * (2 * NBUF),
    )
    def gather(table_hbm, idx_hbm, out_hbm, idx_v, bufs, *sems):
        gsem = sems[:NBUF]
        wsem = sems[NBUF:]
        wid = lax.axis_index("s") * 2 + lax.axis_index("c")
        base = wid * bpw
        pltpu.sync_copy(idx_hbm.at[wid], idx_v)

        def write_chunk(b, j):
            for u in range(BPC):
                pltpu.async_copy(
                    bufs.at[b, pl.ds(u * hist, hist)],
                    out_hbm.at[base + j * BPC + u],
                    wsem[b],
                )

        def wait_write_chunk(b, j):
            for u in range(BPC):
                pltpu.make_async_copy(
                    bufs.at[b, pl.ds(u * hist, hist)],
                    out_hbm.at[base + j * BPC + u],
                    wsem[b],
                ).wait()

        for b in range(LAG):
            pltpu.async_copy(table_hbm.at[idx_v.at[b]], bufs.at[b], gsem[b])

        @pl.loop(0, nch, step=NBUF)
        def _grp(g):
            for b in range(NBUF):
                j = g + b
                jg = j + LAG
                bg = (b + LAG) % NBUF

                @pl.when(jg < nch)
                def _issue_gather():
                    @pl.when(jg >= NBUF)
                    def _wait_write():
                        wait_write_chunk(bg, jg - NBUF)

                    pltpu.async_copy(
                        table_hbm.at[idx_v.at[jg]], bufs.at[bg], gsem[bg]
                    )

                pltpu.make_async_copy(
                    table_hbm.at[idx_v.at[j]], bufs.at[b], gsem[b]
                ).wait()
                write_chunk(b, j)

        for b in range(NBUF):
            wait_write_chunk(b, nch - NBUF + b)

    return gather


def kernel(x, word_embed):
    bat, hist = x.shape
    bpw = bat // NW
    assert bat % NW == 0 and bpw % BPC == 0 and (bpw // BPC) % NBUF == 0
    assert BPC * hist <= 128 and LAG < NBUF
    idx3 = x.reshape(NW, bpw // BPC, BPC * hist).astype(jnp.int32)
    return _make_gather(bat, hist)(word_embed, idx3)


# speedup vs baseline: 3.1872x; 1.7879x over previous
import functools

import jax
import jax.numpy as jnp
from jax import lax
from jax.experimental import pallas as pl
from jax.experimental.pallas import tpu as pltpu
from jax.experimental.pallas import tpu_sc as plsc

D = 128
NW = 32
CHUNK = 128
NBUF = 5
LAG = 2


def _make_gather(bat, hist):
    nch = hist
    mesh = plsc.VectorSubcoreMesh(core_axis_name="c", subcore_axis_name="s")

    @functools.partial(
        pl.kernel,
        out_type=jax.ShapeDtypeStruct((hist, bat, D), jnp.float32),
        mesh=mesh,
        scratch_types=[
            pltpu.VMEM((nch, CHUNK), jnp.int32),
            pltpu.VMEM((NBUF, CHUNK, D), jnp.float32),
        ] + [pltpu.SemaphoreType.DMA] * (2 * NBUF),
    )
    def gather(table_hbm, idx_hbm, out_hbm, idx_v, bufs, *sems):
        gsem = sems[:NBUF]
        wsem = sems[NBUF:]
        wid = lax.axis_index("s") * 2 + lax.axis_index("c")
        col = wid * CHUNK
        pltpu.sync_copy(idx_hbm.at[wid], idx_v)

        for b in range(LAG):
            pltpu.async_copy(table_hbm.at[idx_v.at[b]], bufs.at[b], gsem[b])

        @pl.loop(0, nch, step=NBUF)
        def _grp(g):
            for b in range(NBUF):
                j = g + b
                jg = j + LAG
                bg = (b + LAG) % NBUF

                @pl.when(jg < nch)
                def _issue_gather():
                    @pl.when(jg >= NBUF)
                    def _wait_write():
                        pltpu.make_async_copy(
                            bufs.at[bg],
                            out_hbm.at[jg - NBUF, pl.ds(col, CHUNK)],
                            wsem[bg],
                        ).wait()

                    pltpu.async_copy(
                        table_hbm.at[idx_v.at[jg]], bufs.at[bg], gsem[bg]
                    )

                pltpu.make_async_copy(
                    table_hbm.at[idx_v.at[j]], bufs.at[b], gsem[b]
                ).wait()
                pltpu.async_copy(
                    bufs.at[b], out_hbm.at[j, pl.ds(col, CHUNK)], wsem[b]
                )

        for b in range(NBUF):
            pltpu.make_async_copy(
                bufs.at[b], out_hbm.at[nch - NBUF + b, pl.ds(col, CHUNK)],
                wsem[b],
            ).wait()

    return gather


def kernel(x, word_embed):
    bat, hist = x.shape
    assert bat % (NW * 8) == 0 and bat // NW == CHUNK
    assert hist % NBUF == 0 and LAG < NBUF
    idx3 = jnp.transpose(x, (1, 0)).reshape(hist, NW, CHUNK)
    idx3 = jnp.transpose(idx3, (1, 0, 2)).astype(jnp.int32)
    out = _make_gather(bat, hist)(word_embed, idx3)
    return jnp.transpose(out, (1, 0, 2))


# speedup vs baseline: 3.2035x; 1.0051x over previous
import functools

import jax
import jax.numpy as jnp
from jax import lax
from jax.experimental import pallas as pl
from jax.experimental.pallas import tpu as pltpu
from jax.experimental.pallas import tpu_sc as plsc

D = 128
NW = 32
CHUNK = 128
NBUF = 5
LAG = 3


def _make_gather(bat, hist):
    nch = hist
    mesh = plsc.VectorSubcoreMesh(core_axis_name="c", subcore_axis_name="s")

    @functools.partial(
        pl.kernel,
        out_type=jax.ShapeDtypeStruct((hist, bat, D), jnp.float32),
        mesh=mesh,
        scratch_types=[
            pltpu.VMEM((nch, CHUNK), jnp.int32),
            pltpu.VMEM((NBUF, CHUNK, D), jnp.float32),
        ] + [pltpu.SemaphoreType.DMA] * (2 * NBUF),
    )
    def gather(table_hbm, idx_hbm, out_hbm, idx_v, bufs, *sems):
        gsem = sems[:NBUF]
        wsem = sems[NBUF:]
        wid = lax.axis_index("s") * 2 + lax.axis_index("c")
        col = wid * CHUNK
        pltpu.sync_copy(idx_hbm.at[wid], idx_v)

        for b in range(LAG):
            pltpu.async_copy(table_hbm.at[idx_v.at[b]], bufs.at[b], gsem[b])

        @pl.loop(0, nch, step=NBUF)
        def _grp(g):
            for b in range(NBUF):
                j = g + b
                jg = j + LAG
                bg = (b + LAG) % NBUF

                @pl.when(jg < nch)
                def _issue_gather():
                    @pl.when(jg >= NBUF)
                    def _wait_write():
                        pltpu.make_async_copy(
                            bufs.at[bg],
                            out_hbm.at[jg - NBUF, pl.ds(col, CHUNK)],
                            wsem[bg],
                        ).wait()

                    pltpu.async_copy(
                        table_hbm.at[idx_v.at[jg]], bufs.at[bg], gsem[bg]
                    )

                pltpu.make_async_copy(
                    table_hbm.at[idx_v.at[j]], bufs.at[b], gsem[b]
                ).wait()
                pltpu.async_copy(
                    bufs.at[b], out_hbm.at[j, pl.ds(col, CHUNK)], wsem[b]
                )

        for b in range(NBUF):
            pltpu.make_async_copy(
                bufs.at[b], out_hbm.at[nch - NBUF + b, pl.ds(col, CHUNK)],
                wsem[b],
            ).wait()

    return gather


def kernel(x, word_embed):
    bat, hist = x.shape
    assert bat % (NW * 8) == 0 and bat // NW == CHUNK
    assert hist % NBUF == 0 and LAG < NBUF
    idx3 = jnp.transpose(x, (1, 0)).reshape(hist, NW, CHUNK)
    idx3 = jnp.transpose(idx3, (1, 0, 2)).astype(jnp.int32)
    out = _make_gather(bat, hist)(word_embed, idx3)
    return jnp.transpose(out, (1, 0, 2))


# speedup vs baseline: 3.2066x; 1.0010x over previous
import functools

import jax
import jax.numpy as jnp
from jax import lax
from jax.experimental import pallas as pl
from jax.experimental.pallas import tpu as pltpu
from jax.experimental.pallas import tpu_sc as plsc

D = 128
NW = 32
CHUNK = 128
NBUF = 5
LAG = 4


def _make_gather(bat, hist):
    nch = hist
    mesh = plsc.VectorSubcoreMesh(core_axis_name="c", subcore_axis_name="s")

    @functools.partial(
        pl.kernel,
        out_type=jax.ShapeDtypeStruct((hist, bat, D), jnp.float32),
        mesh=mesh,
        scratch_types=[
            pltpu.VMEM((nch, CHUNK), jnp.int32),
            pltpu.VMEM((NBUF, CHUNK, D), jnp.float32),
        ] + [pltpu.SemaphoreType.DMA] * (2 * NBUF),
    )
    def gather(table_hbm, idx_hbm, out_hbm, idx_v, bufs, *sems):
        gsem = sems[:NBUF]
        wsem = sems[NBUF:]
        wid = lax.axis_index("s") * 2 + lax.axis_index("c")
        col = wid * CHUNK
        pltpu.sync_copy(idx_hbm.at[wid], idx_v)

        for b in range(LAG):
            pltpu.async_copy(table_hbm.at[idx_v.at[b]], bufs.at[b], gsem[b])

        @pl.loop(0, nch, step=NBUF)
        def _grp(g):
            for b in range(NBUF):
                j = g + b
                jg = j + LAG
                bg = (b + LAG) % NBUF

                @pl.when(jg < nch)
                def _issue_gather():
                    @pl.when(jg >= NBUF)
                    def _wait_write():
                        pltpu.make_async_copy(
                            bufs.at[bg],
                            out_hbm.at[jg - NBUF, pl.ds(col, CHUNK)],
                            wsem[bg],
                        ).wait()

                    pltpu.async_copy(
                        table_hbm.at[idx_v.at[jg]], bufs.at[bg], gsem[bg]
                    )

                pltpu.make_async_copy(
                    table_hbm.at[idx_v.at[j]], bufs.at[b], gsem[b]
                ).wait()
                pltpu.async_copy(
                    bufs.at[b], out_hbm.at[j, pl.ds(col, CHUNK)], wsem[b]
                )

        for b in range(NBUF):
            pltpu.make_async_copy(
                bufs.at[b], out_hbm.at[nch - NBUF + b, pl.ds(col, CHUNK)],
                wsem[b],
            ).wait()

    return gather


def kernel(x, word_embed):
    bat, hist = x.shape
    assert bat % (NW * 8) == 0 and bat // NW == CHUNK
    assert hist % NBUF == 0 and LAG < NBUF
    idx3 = jnp.transpose(x, (1, 0)).reshape(hist, NW, CHUNK)
    idx3 = jnp.transpose(idx3, (1, 0, 2)).astype(jnp.int32)
    out = _make_gather(bat, hist)(word_embed, idx3)
    return jnp.transpose(out, (1, 0, 2))
